# baseline (speedup 1.0000x reference)
import functools

import jax
import jax.numpy as jnp
from jax import lax
from jax.experimental import pallas as pl
from jax.experimental.pallas import tpu as pltpu
from jax.experimental.pallas import tpu_sc as plsc

NC = 2
NS = 16
CHUNK = 384

ROWS = 2000


def _relu(v):
    return jnp.maximum(v, 0.0)


def _dot(a, b):
    return jax.lax.dot_general(a, b, (((1,), (0,)), ((), ())),
                               preferred_element_type=jnp.float32)


def _pack(m):
    q = ROWS // 4
    halves = []
    for h in (0, 1):
        mh = m[:, 32 * h:32 * h + 32]
        halves.append(jnp.concatenate(
            [mh[q * g:q * g + q] for g in range(4)], axis=1))
    return jnp.concatenate(halves, axis=0)


def _unpack(a):
    q = ROWS // 4
    cols = []
    for h in (0, 1):
        ah = a[q * h:q * h + q]
        cols.append(jnp.concatenate(
            [ah[:, 32 * g:32 * g + 32] for g in range(4)], axis=0))
    return jnp.concatenate(cols, axis=1)


def _enc_body(x_ref, w0, b0, w1, b1, w2, b2, cw, out_ref):
    h = _relu(_dot(x_ref[...], w0[...]) + b0[...])
    h = _relu(_dot(h, w1[...]) + b1[...])
    h = _relu(_dot(h, w2[...]) + b2[...])
    out_ref[...] = _pack(_dot(h, cw[...]))


def _mid_body(a_ref, b_ref, cw, out_ref):
    h = _relu(_unpack(a_ref[...]) + b_ref[...])
    out_ref[...] = _pack(_dot(h, cw[...]))


def _dec_body(a_ref, cb, w0, b0, w1, b1, w2, b2, out_ref):
    h = _relu(_unpack(a_ref[...]) + cb[...])
    h = _relu(_dot(h, w0[...]) + b0[...])
    h = _relu(_dot(h, w1[...]) + b1[...])
    out_ref[...] = _relu(_dot(h, w2[...]) + b2[...])


def _full(shape):
    return pl.BlockSpec(shape, lambda b: (0,) * len(shape))


def _enc_call(n):
    grid = n // ROWS
    return pl.pallas_call(
        _enc_body,
        grid=(grid,),
        in_specs=[
            pl.BlockSpec((ROWS, 128), lambda b: (b, 0)),
            _full((128, 128)), _full((1, 128)),
            _full((128, 96)), _full((1, 96)),
            _full((96, 64)), _full((1, 64)),
            _full((64, 64)),
        ],
        out_specs=pl.BlockSpec((ROWS // 2, 128), lambda b: (b, 0)),
        out_shape=jax.ShapeDtypeStruct((n // 2, 128), jnp.float32),
    )


def _mid_call(n):
    grid = n // ROWS
    return pl.pallas_call(
        _mid_body,
        grid=(grid,),
        in_specs=[
            pl.BlockSpec((ROWS // 2, 128), lambda b: (b, 0)),
            _full((1, 64)),
            _full((64, 64)),
        ],
        out_specs=pl.BlockSpec((ROWS // 2, 128), lambda b: (b, 0)),
        out_shape=jax.ShapeDtypeStruct((n // 2, 128), jnp.float32),
    )


def _dec_call(n):
    grid = n // ROWS
    return pl.pallas_call(
        _dec_body,
        grid=(grid,),
        in_specs=[
            pl.BlockSpec((ROWS // 2, 128), lambda b: (b, 0)),
            _full((1, 64)),
            _full((64, 96)), _full((1, 96)),
            _full((96, 128)), _full((1, 128)),
            _full((128, 128)), _full((1, 128)),
        ],
        out_specs=pl.BlockSpec((ROWS, 128), lambda b: (b, 0)),
        out_shape=jax.ShapeDtypeStruct((n, 128), jnp.float32),
    )


def _make_scatter(n, ep):
    cpt = ep // (NS * CHUNK)
    assert cpt % 4 == 0 and cpt >= 8
    zr = (-(-(n + 1) // NS) + 7) // 8 * 8
    np_rows = zr * NS
    nblk = n // ROWS
    assert n % ROWS == 0
    mesh = plsc.VectorSubcoreMesh(core_axis_name="c", subcore_axis_name="s",
                                  num_cores=NC, num_subcores=NS)

    @functools.partial(
        pl.kernel,
        out_type=jax.ShapeDtypeStruct((2 * n, 32), jnp.float32),
        mesh=mesh,
        compiler_params=pltpu.CompilerParams(use_tc_tiling_on_sc=False),
        scratch_types=[
            pltpu.VMEM((CHUNK,), jnp.int32),
            pltpu.VMEM((CHUNK,), jnp.int32),
            pltpu.VMEM((CHUNK,), jnp.int32),
            pltpu.VMEM((CHUNK,), jnp.int32),
            pltpu.VMEM((CHUNK,), jnp.int32),
            pltpu.VMEM((CHUNK,), jnp.int32),
            pltpu.VMEM((CHUNK, 32), jnp.float32),
            pltpu.VMEM((CHUNK, 32), jnp.float32),
            pltpu.VMEM_SHARED((np_rows, 32), jnp.float32),
            pltpu.SemaphoreType.DMA,
            pltpu.SemaphoreType.DMA,
            pltpu.SemaphoreType.DMA,
            pltpu.SemaphoreType.DMA,
            pltpu.SemaphoreType.DMA,
            pltpu.SemaphoreType.DMA,
            pltpu.SemaphoreType.DMA,
            pltpu.SemaphoreType.DMA,
            pltpu.SemaphoreType.DMA,
            pltpu.SemaphoreType.DMA,
        ],
    )
    def scatter(mtab, src2, dst2, zst, out, sb0, sb1, db0, db1, db2, db3,
                rb0, rb1, acc, ss0, ss1, sd0, sd1, sd2, sd3,
                sg0, sg1, sc0, sc1):
        c = lax.axis_index("c")
        s = lax.axis_index("s")
        sbufs, rbufs = (sb0, sb1), (rb0, rb1)
        dbufs = (db0, db1, db2, db3)
        ssems, gsems, csems = (ss0, ss1), (sg0, sg1), (sc0, sc1)
        dsems = (sd0, sd1, sd2, sd3)
        ebase = s * (cpt * CHUNK)

        pltpu.sync_copy(zst, acc.at[pl.ds(s * zr, zr)])
        plsc.subcore_barrier()

        def start_idx(u, b, d):
            pltpu.async_copy(src2.at[c, pl.ds(ebase + u * CHUNK, CHUNK)],
                             sbufs[b], ssems[b])
            pltpu.async_copy(dst2.at[pl.ds(ebase + u * CHUNK, CHUNK)],
                             dbufs[d], dsems[d])

        def start_gather(b):
            pltpu.async_copy(mtab.at[sbufs[b]], rbufs[b], gsems[b])

        def wait_idx(b, d):
            pltpu.make_async_copy(src2.at[c, pl.ds(0, CHUNK)],
                                  sbufs[b], ssems[b]).wait()
            pltpu.make_async_copy(dst2.at[pl.ds(0, CHUNK)],
                                  dbufs[d], dsems[d]).wait()

        def wait_gather(b):
            pltpu.make_async_copy(mtab.at[sbufs[b]], rbufs[b],
                                  gsems[b]).wait()

        def start_scatter(b, d):
            pltpu.async_copy(rbufs[b], acc.at[dbufs[d]], csems[b], add=True)

        def drain_scatter(b, d):
            pltpu.make_async_copy(rbufs[b], acc.at[dbufs[d]],
                                  csems[b]).wait()

        start_idx(0, 0, 0)
        start_idx(1, 1, 1)
        wait_idx(0, 0)
        start_gather(0)

        def group_body(g, carry):
            for q in range(4):
                b = q % 2
                nb = 1 - b

                u = 4 * g + q

                @pl.when(u >= 1)
                def _drain_prev():
                    drain_scatter(nb, (q + 3) % 4)

                @pl.when(u < cpt - 1)
                def _gather_next():
                    wait_idx(nb, (q + 1) % 4)
                    start_gather(nb)

                wait_gather(b)
                start_scatter(b, q % 4)

                @pl.when(u < cpt - 2)
                def _idx_next():
                    start_idx(u + 2, b, (q + 2) % 4)
            return carry

        lax.fori_loop(0, cpt // 4, group_body, 0)
        drain_scatter((cpt - 1) % 2, (cpt - 1) % 4)
        plsc.subcore_barrier()

        def wb(bb):
            pltpu.sync_copy(acc.at[pl.ds(bb * ROWS, ROWS)],
                            out.at[pl.ds(2 * ROWS * bb + ROWS * c, ROWS)])

        @pl.when(s < nblk)
        def _wb0():
            wb(s)

        @pl.when(s + NS < nblk)
        def _wb1():
            wb(s + NS)

    return scatter


def kernel(x, edge_index, enc_W0, enc_b0, enc_W1, enc_b1, enc_W2, enc_b2,
           conv_W0, conv_b0, conv_W1, conv_b1,
           dec_W0, dec_b0, dec_W1, dec_b1, dec_W2, dec_b2):
    n = x.shape[0]
    e = edge_index.shape[1]
    cpt = (-(-e // (NS * CHUNK)) + 3) // 4 * 4
    ep = cpt * NS * CHUNK
    zr = (-(-(n + 1) // NS) + 7) // 8 * 8

    src = edge_index[0]
    dst = edge_index[1]
    pad = ep - e
    q = ROWS // 4

    def sig(v):
        l = v % ROWS
        return 4 * (l % q) + l // q

    src_p = jnp.concatenate([src, jnp.zeros((pad,), jnp.int32)])
    base = 2 * ROWS * (src_p // ROWS) + sig(src_p)
    src2 = jnp.stack([base, base + ROWS])
    dst_a = ROWS * (dst // ROWS) + sig(dst)
    dst2 = jnp.concatenate([dst_a, jnp.full((pad,), n, jnp.int32)])
    zst = jnp.zeros((zr, 32), jnp.float32)

    def r2(b):
        return b.reshape(1, -1)

    scatter = _make_scatter(n, ep)

    m0 = _enc_call(n)(x, enc_W0, r2(enc_b0), enc_W1, r2(enc_b1),
                      enc_W2, r2(enc_b2), conv_W0)
    agg0 = scatter(m0.reshape(2 * n, 32), src2, dst2, zst)
    m1 = _mid_call(n)(agg0.reshape(n // 2, 128), r2(conv_b0), conv_W1)
    agg1 = scatter(m1.reshape(2 * n, 32), src2, dst2, zst)
    out = _dec_call(n)(agg1.reshape(n // 2, 128), r2(conv_b1),
                       dec_W0, r2(dec_b0), dec_W1, r2(dec_b1),
                       dec_W2, r2(dec_b2))
    return out


# speedup vs baseline: 1.5155x; 1.5155x over previous
import functools

import jax
import jax.numpy as jnp
from jax import lax
from jax.experimental import pallas as pl
from jax.experimental.pallas import tpu as pltpu
from jax.experimental.pallas import tpu_sc as plsc

NC = 2
NS = 16
CHUNK = 432

ROWS = 2000


def _relu(v):
    return jnp.maximum(v, 0.0)


def _dot(a, b):
    return jax.lax.dot_general(a, b, (((1,), (0,)), ((), ())),
                               preferred_element_type=jnp.float32)


def _pack(m):
    q = ROWS // 4
    halves = []
    for h in (0, 1):
        mh = m[:, 32 * h:32 * h + 32]
        halves.append(jnp.concatenate(
            [mh[q * g:q * g + q] for g in range(4)], axis=1))
    return jnp.concatenate(halves, axis=0)


def _unpack(a):
    q = ROWS // 4
    cols = []
    for h in (0, 1):
        ah = a[q * h:q * h + q]
        cols.append(jnp.concatenate(
            [ah[:, 32 * g:32 * g + 32] for g in range(4)], axis=0))
    return jnp.concatenate(cols, axis=1)


def _enc_body(x_ref, w0, b0, w1, b1, w2, b2, cw, out_ref):
    h = _relu(_dot(x_ref[...], w0[...]) + b0[...])
    h = _relu(_dot(h, w1[...]) + b1[...])
    h = _relu(_dot(h, w2[...]) + b2[...])
    out_ref[...] = _pack(_dot(h, cw[...]))


def _mid_body(a_ref, b_ref, cw, out_ref):
    h = _relu(_unpack(a_ref[...]) + b_ref[...])
    out_ref[...] = _pack(_dot(h, cw[...]))


def _dec_body(a_ref, cb, w0, b0, w1, b1, w2, b2, out_ref):
    h = _relu(_unpack(a_ref[...]) + cb[...])
    h = _relu(_dot(h, w0[...]) + b0[...])
    h = _relu(_dot(h, w1[...]) + b1[...])
    out_ref[...] = _relu(_dot(h, w2[...]) + b2[...])


def _full(shape):
    return pl.BlockSpec(shape, lambda b: (0,) * len(shape))


def _enc_call(n):
    grid = n // ROWS
    return pl.pallas_call(
        _enc_body,
        grid=(grid,),
        in_specs=[
            pl.BlockSpec((ROWS, 128), lambda b: (b, 0)),
            _full((128, 128)), _full((1, 128)),
            _full((128, 96)), _full((1, 96)),
            _full((96, 64)), _full((1, 64)),
            _full((64, 64)),
        ],
        out_specs=pl.BlockSpec((ROWS // 2, 128), lambda b: (b, 0)),
        out_shape=jax.ShapeDtypeStruct((n // 2, 128), jnp.float32),
    )


def _mid_call(n):
    grid = n // ROWS
    return pl.pallas_call(
        _mid_body,
        grid=(grid,),
        in_specs=[
            pl.BlockSpec((ROWS // 2, 128), lambda b: (b, 0)),
            _full((1, 64)),
            _full((64, 64)),
        ],
        out_specs=pl.BlockSpec((ROWS // 2, 128), lambda b: (b, 0)),
        out_shape=jax.ShapeDtypeStruct((n // 2, 128), jnp.float32),
    )


def _dec_call(n):
    grid = n // ROWS
    return pl.pallas_call(
        _dec_body,
        grid=(grid,),
        in_specs=[
            pl.BlockSpec((ROWS // 2, 128), lambda b: (b, 0)),
            _full((1, 64)),
            _full((64, 96)), _full((1, 96)),
            _full((96, 128)), _full((1, 128)),
            _full((128, 128)), _full((1, 128)),
        ],
        out_specs=pl.BlockSpec((ROWS, 128), lambda b: (b, 0)),
        out_shape=jax.ShapeDtypeStruct((n, 128), jnp.float32),
    )


def _make_scatter(n, ep):
    cpt = ep // (NS * CHUNK)
    assert cpt % 4 == 0 and cpt >= 8
    zr = (-(-(n + 1) // NS) + 7) // 8 * 8
    np_rows = zr * NS
    nblk = n // ROWS
    assert n % ROWS == 0
    mesh = plsc.VectorSubcoreMesh(core_axis_name="c", subcore_axis_name="s",
                                  num_cores=NC, num_subcores=NS)

    @functools.partial(
        pl.kernel,
        out_type=jax.ShapeDtypeStruct((2 * n, 32), jnp.float32),
        mesh=mesh,
        compiler_params=pltpu.CompilerParams(use_tc_tiling_on_sc=False),
        scratch_types=[
            pltpu.VMEM((CHUNK,), jnp.int32),
            pltpu.VMEM((CHUNK,), jnp.int32),
            pltpu.VMEM((CHUNK,), jnp.int32),
            pltpu.VMEM((CHUNK,), jnp.int32),
            pltpu.VMEM((CHUNK,), jnp.int32),
            pltpu.VMEM((CHUNK,), jnp.int32),
            pltpu.VMEM((CHUNK, 32), jnp.float32),
            pltpu.VMEM((CHUNK, 32), jnp.float32),
            pltpu.VMEM_SHARED((np_rows, 32), jnp.float32),
            pltpu.SemaphoreType.DMA,
            pltpu.SemaphoreType.DMA,
            pltpu.SemaphoreType.DMA,
            pltpu.SemaphoreType.DMA,
            pltpu.SemaphoreType.DMA,
            pltpu.SemaphoreType.DMA,
            pltpu.SemaphoreType.DMA,
            pltpu.SemaphoreType.DMA,
            pltpu.SemaphoreType.DMA,
            pltpu.SemaphoreType.DMA,
        ],
    )
    def scatter(mtab, src1, dst2, zst, out, sb0, sb1, db0, db1, db2, db3,
                rb0, rb1, acc, ss0, ss1, sd0, sd1, sd2, sd3,
                sg0, sg1, sc0, sc1):
        c = lax.axis_index("c")
        s = lax.axis_index("s")
        sbufs, rbufs = (sb0, sb1), (rb0, rb1)
        dbufs = (db0, db1, db2, db3)
        ssems, gsems, csems = (ss0, ss1), (sg0, sg1), (sc0, sc1)
        dsems = (sd0, sd1, sd2, sd3)
        ebase = s * (cpt * CHUNK)
        coff = jax.lax.broadcast(c * ROWS, (16,))

        def start_idx(u, b, d):
            pltpu.async_copy(src1.at[pl.ds(ebase + u * CHUNK, CHUNK)],
                             sbufs[b], ssems[b])
            pltpu.async_copy(dst2.at[pl.ds(ebase + u * CHUNK, CHUNK)],
                             dbufs[d], dsems[d])

        def start_gather(b):
            pltpu.async_copy(mtab.at[sbufs[b]], rbufs[b], gsems[b])

        def wait_idx(b, d):
            pltpu.make_async_copy(src1.at[pl.ds(0, CHUNK)],
                                  sbufs[b], ssems[b]).wait()
            pltpu.make_async_copy(dst2.at[pl.ds(0, CHUNK)],
                                  dbufs[d], dsems[d]).wait()
            for k in range(CHUNK // 16):
                sbufs[b][pl.ds(16 * k, 16)] = (
                    sbufs[b][pl.ds(16 * k, 16)] + coff)

        def wait_gather(b):
            pltpu.make_async_copy(mtab.at[sbufs[b]], rbufs[b],
                                  gsems[b]).wait()

        def start_scatter(b, d):
            pltpu.async_copy(rbufs[b], acc.at[dbufs[d]], csems[b], add=True)

        def drain_scatter(b, d):
            pltpu.make_async_copy(rbufs[b], acc.at[dbufs[d]],
                                  csems[b]).wait()

        start_idx(0, 0, 0)
        start_idx(1, 1, 1)
        wait_idx(0, 0)
        start_gather(0)
        pltpu.sync_copy(zst, acc.at[pl.ds(s * zr, zr)])
        plsc.subcore_barrier()

        def group_body(g, carry):
            for q in range(4):
                b = q % 2
                nb = 1 - b

                u = 4 * g + q

                @pl.when(u >= 1)
                def _drain_prev():
                    drain_scatter(nb, (q + 3) % 4)

                @pl.when(u < cpt - 1)
                def _gather_next():
                    wait_idx(nb, (q + 1) % 4)
                    start_gather(nb)

                wait_gather(b)
                start_scatter(b, q % 4)

                @pl.when(u < cpt - 2)
                def _idx_next():
                    start_idx(u + 2, b, (q + 2) % 4)
            return carry

        lax.fori_loop(0, cpt // 4, group_body, 0)
        drain_scatter((cpt - 1) % 2, (cpt - 1) % 4)
        plsc.subcore_barrier()

        def wb(bb):
            pltpu.sync_copy(acc.at[pl.ds(bb * ROWS, ROWS)],
                            out.at[pl.ds(2 * ROWS * bb + ROWS * c, ROWS)])

        @pl.when(s < nblk)
        def _wb0():
            wb(s)

        @pl.when(s + NS < nblk)
        def _wb1():
            wb(s + NS)

    return scatter


def kernel(x, edge_index, enc_W0, enc_b0, enc_W1, enc_b1, enc_W2, enc_b2,
           conv_W0, conv_b0, conv_W1, conv_b1,
           dec_W0, dec_b0, dec_W1, dec_b1, dec_W2, dec_b2):
    n = x.shape[0]
    e = edge_index.shape[1]
    cpt = (-(-e // (NS * CHUNK)) + 3) // 4 * 4
    ep = cpt * NS * CHUNK
    zr = (-(-(n + 1) // NS) + 7) // 8 * 8

    src = edge_index[0]
    dst = edge_index[1]
    pad = ep - e
    q = ROWS // 4

    def sig(v):
        l = v % ROWS
        return 4 * (l % q) + l // q

    src_p = jnp.concatenate([src, jnp.zeros((pad,), jnp.int32)])
    src1 = 2 * ROWS * (src_p // ROWS) + sig(src_p)
    dst_a = ROWS * (dst // ROWS) + sig(dst)
    dst2 = jnp.concatenate([dst_a, jnp.full((pad,), n, jnp.int32)])
    zst = jnp.zeros((zr, 32), jnp.float32)

    def r2(b):
        return b.reshape(1, -1)

    scatter = _make_scatter(n, ep)

    m0 = _enc_call(n)(x, enc_W0, r2(enc_b0), enc_W1, r2(enc_b1),
                      enc_W2, r2(enc_b2), conv_W0)
    agg0 = scatter(m0.reshape(2 * n, 32), src1, dst2, zst)
    m1 = _mid_call(n)(agg0.reshape(n // 2, 128), r2(conv_b0), conv_W1)
    agg1 = scatter(m1.reshape(2 * n, 32), src1, dst2, zst)
    out = _dec_call(n)(agg1.reshape(n // 2, 128), r2(conv_b1),
                       dec_W0, r2(dec_b0), dec_W1, r2(dec_b1),
                       dec_W2, r2(dec_b2))
    return out


# speedup vs baseline: 1.6647x; 1.0984x over previous
import functools

import jax
import jax.numpy as jnp
from jax import lax
from jax.experimental import pallas as pl
from jax.experimental.pallas import tpu as pltpu
from jax.experimental.pallas import tpu_sc as plsc

NC = 2
NS = 16
CHUNK = 432

ROWS = 2000


def _relu(v):
    return jnp.maximum(v, 0.0)


def _dot(a, b):
    return jax.lax.dot_general(a, b, (((1,), (0,)), ((), ())),
                               preferred_element_type=jnp.float32)


def _pack(m):
    q = ROWS // 4
    halves = []
    for h in (0, 1):
        mh = m[:, 32 * h:32 * h + 32]
        halves.append(jnp.concatenate(
            [mh[q * g:q * g + q] for g in range(4)], axis=1))
    return jnp.concatenate(halves, axis=0)


def _unpack(a):
    q = ROWS // 4
    cols = []
    for h in (0, 1):
        ah = a[q * h:q * h + q]
        cols.append(jnp.concatenate(
            [ah[:, 32 * g:32 * g + 32] for g in range(4)], axis=0))
    return jnp.concatenate(cols, axis=1)


def _enc_body(x_ref, w0, b0, w1, b1, w2, b2, cw, out_ref):
    h = _relu(_dot(x_ref[...], w0[...]) + b0[...])
    h = _relu(_dot(h, w1[...]) + b1[...])
    h = _relu(_dot(h, w2[...]) + b2[...])
    out_ref[...] = _pack(_dot(h, cw[...]))


def _mid_body(a_ref, b_ref, cw, out_ref):
    h = _relu(_unpack(a_ref[...]) + b_ref[...])
    out_ref[...] = _pack(_dot(h, cw[...]))


def _dec_body(a_ref, cb, w0, b0, w1, b1, w2, b2, out_ref):
    h = _relu(_unpack(a_ref[...]) + cb[...])
    h = _relu(_dot(h, w0[...]) + b0[...])
    h = _relu(_dot(h, w1[...]) + b1[...])
    out_ref[...] = _relu(_dot(h, w2[...]) + b2[...])


def _full(shape):
    return pl.BlockSpec(shape, lambda b: (0,) * len(shape))


def _enc_call(n):
    grid = n // ROWS
    return pl.pallas_call(
        _enc_body,
        grid=(grid,),
        in_specs=[
            pl.BlockSpec((ROWS, 128), lambda b: (b, 0)),
            _full((128, 128)), _full((1, 128)),
            _full((128, 96)), _full((1, 96)),
            _full((96, 64)), _full((1, 64)),
            _full((64, 64)),
        ],
        out_specs=pl.BlockSpec((ROWS // 2, 128), lambda b: (b, 0)),
        out_shape=jax.ShapeDtypeStruct((n // 2, 128), jnp.float32),
    )


def _mid_call(n):
    grid = n // ROWS
    return pl.pallas_call(
        _mid_body,
        grid=(grid,),
        in_specs=[
            pl.BlockSpec((ROWS // 2, 128), lambda b: (b, 0)),
            _full((1, 64)),
            _full((64, 64)),
        ],
        out_specs=pl.BlockSpec((ROWS // 2, 128), lambda b: (b, 0)),
        out_shape=jax.ShapeDtypeStruct((n // 2, 128), jnp.float32),
    )


def _dec_call(n):
    grid = n // ROWS
    return pl.pallas_call(
        _dec_body,
        grid=(grid,),
        in_specs=[
            pl.BlockSpec((ROWS // 2, 128), lambda b: (b, 0)),
            _full((1, 64)),
            _full((64, 96)), _full((1, 96)),
            _full((96, 128)), _full((1, 128)),
            _full((128, 128)), _full((1, 128)),
        ],
        out_specs=pl.BlockSpec((ROWS, 128), lambda b: (b, 0)),
        out_shape=jax.ShapeDtypeStruct((n, 128), jnp.float32),
    )


def _make_scatter(n, ep):
    cpt = ep // (NS * CHUNK)
    assert cpt % 4 == 0 and cpt >= 8
    zr = (-(-(n + 1) // NS) + 7) // 8 * 8
    np_rows = zr * NS
    nblk = n // ROWS
    assert n % ROWS == 0
    mesh = plsc.VectorSubcoreMesh(core_axis_name="c", subcore_axis_name="s",
                                  num_cores=NC, num_subcores=NS)

    @functools.partial(
        pl.kernel,
        out_type=jax.ShapeDtypeStruct((2 * n, 32), jnp.float32),
        mesh=mesh,
        compiler_params=pltpu.CompilerParams(use_tc_tiling_on_sc=False),
        scratch_types=[
            pltpu.VMEM((CHUNK,), jnp.int32),
            pltpu.VMEM((CHUNK,), jnp.int32),
            pltpu.VMEM((CHUNK,), jnp.int32),
            pltpu.VMEM((CHUNK,), jnp.int32),
            pltpu.VMEM((CHUNK,), jnp.int32),
            pltpu.VMEM((CHUNK,), jnp.int32),
            pltpu.VMEM((CHUNK, 32), jnp.float32),
            pltpu.VMEM((CHUNK, 32), jnp.float32),
            pltpu.VMEM_SHARED((np_rows, 32), jnp.float32),
            pltpu.SemaphoreType.DMA,
            pltpu.SemaphoreType.DMA,
            pltpu.SemaphoreType.DMA,
            pltpu.SemaphoreType.DMA,
            pltpu.SemaphoreType.DMA,
            pltpu.SemaphoreType.DMA,
            pltpu.SemaphoreType.DMA,
            pltpu.SemaphoreType.DMA,
            pltpu.SemaphoreType.DMA,
            pltpu.SemaphoreType.DMA,
        ],
    )
    def scatter(mtab, src1, dst2, zst, out, sb0, sb1, db0, db1, db2, db3,
                rb0, rb1, acc, ss0, ss1, sd0, sd1, sd2, sd3,
                sg0, sg1, sc0, sc1):
        c = lax.axis_index("c")
        s = lax.axis_index("s")
        sbufs, rbufs = (sb0, sb1), (rb0, rb1)
        dbufs = (db0, db1, db2, db3)
        ssems, gsems, csems = (ss0, ss1), (sg0, sg1), (sc0, sc1)
        dsems = (sd0, sd1, sd2, sd3)
        ebase = s * (cpt * CHUNK)
        coff = jax.lax.broadcast(c * ROWS, (16,))

        def start_idx(u, b, d):
            pltpu.async_copy(src1.at[pl.ds(ebase + u * CHUNK, CHUNK)],
                             sbufs[b], ssems[b])
            pltpu.async_copy(dst2.at[pl.ds(ebase + u * CHUNK, CHUNK)],
                             dbufs[d], dsems[d])

        def start_gather(b):
            pltpu.async_copy(mtab.at[sbufs[b]], rbufs[b], gsems[b])

        def wait_idx(b, d):
            pltpu.make_async_copy(src1.at[pl.ds(0, CHUNK)],
                                  sbufs[b], ssems[b]).wait()
            pltpu.make_async_copy(dst2.at[pl.ds(0, CHUNK)],
                                  dbufs[d], dsems[d]).wait()
            for k in range(CHUNK // 16):
                sbufs[b][pl.ds(16 * k, 16)] = (
                    sbufs[b][pl.ds(16 * k, 16)] + coff)

        def wait_gather(b):
            pltpu.make_async_copy(mtab.at[sbufs[b]], rbufs[b],
                                  gsems[b]).wait()

        def start_scatter(b, d):
            pltpu.async_copy(rbufs[b], acc.at[dbufs[d]], csems[b], add=True)

        def drain_scatter(b, d):
            pltpu.make_async_copy(rbufs[b], acc.at[dbufs[d]],
                                  csems[b]).wait()

        start_idx(0, 0, 0)
        start_idx(1, 1, 1)
        wait_idx(0, 0)
        start_gather(0)
        pltpu.sync_copy(zst, acc.at[pl.ds(s * zr, zr)])
        plsc.subcore_barrier()

        def group_body(g, carry):
            for q in range(4):
                b = q % 2
                nb = 1 - b

                u = 4 * g + q

                @pl.when(u >= 1)
                def _drain_prev():
                    drain_scatter(nb, (q + 3) % 4)

                @pl.when(u < cpt - 1)
                def _gather_next():
                    wait_idx(nb, (q + 1) % 4)
                    start_gather(nb)

                wait_gather(b)
                start_scatter(b, q % 4)

                @pl.when(u < cpt - 2)
                def _idx_next():
                    start_idx(u + 2, b, (q + 2) % 4)
            return carry

        lax.fori_loop(0, cpt // 4, group_body, 0)
        drain_scatter((cpt - 1) % 2, (cpt - 1) % 4)
        plsc.subcore_barrier()

        def wb(bb):
            pltpu.sync_copy(acc.at[pl.ds(bb * ROWS, ROWS)],
                            out.at[pl.ds(2 * ROWS * bb + ROWS * c, ROWS)])

        @pl.when(s < nblk)
        def _wb0():
            wb(s)

        @pl.when(s + NS < nblk)
        def _wb1():
            wb(s + NS)

    return scatter


def kernel(x, edge_index, enc_W0, enc_b0, enc_W1, enc_b1, enc_W2, enc_b2,
           conv_W0, conv_b0, conv_W1, conv_b1,
           dec_W0, dec_b0, dec_W1, dec_b1, dec_W2, dec_b2):
    n = x.shape[0]
    e = edge_index.shape[1]
    cpt = (-(-e // (NS * CHUNK)) + 3) // 4 * 4
    ep = cpt * NS * CHUNK
    zr = (-(-(n + 1) // NS) + 7) // 8 * 8

    src = edge_index[0]
    dst = edge_index[1]
    pad = ep - e
    def packed_row(v, half_stride):
        blk = ((v >> 4) * 4195) >> 19
        l = v - ROWS * blk
        g = ((l >> 2) * 4195) >> 19
        p = l - (ROWS // 4) * g
        return half_stride * blk + 4 * p + g

    src_p = jnp.concatenate([src, jnp.zeros((pad,), jnp.int32)])
    src1 = packed_row(src_p, 2 * ROWS)
    dst_p = jnp.concatenate([dst, jnp.full((pad,), n, jnp.int32)])
    dst2 = packed_row(dst_p, ROWS)
    zst = jnp.zeros((zr, 32), jnp.float32)

    def r2(b):
        return b.reshape(1, -1)

    scatter = _make_scatter(n, ep)

    m0 = _enc_call(n)(x, enc_W0, r2(enc_b0), enc_W1, r2(enc_b1),
                      enc_W2, r2(enc_b2), conv_W0)
    agg0 = scatter(m0.reshape(2 * n, 32), src1, dst2, zst)
    m1 = _mid_call(n)(agg0.reshape(n // 2, 128), r2(conv_b0), conv_W1)
    agg1 = scatter(m1.reshape(2 * n, 32), src1, dst2, zst)
    out = _dec_call(n)(agg1.reshape(n // 2, 128), r2(conv_b1),
                       dec_W0, r2(dec_b0), dec_W1, r2(dec_b1),
                       dec_W2, r2(dec_b2))
    return out
